# baseline (speedup 1.0000x reference)
import functools

import jax
import jax.numpy as jnp
from jax import lax
from jax.experimental import pallas as pl
from jax.experimental.pallas import tpu as pltpu
from jax.experimental.pallas import tpu_sc as plsc

B = 16384
D = 64
NEG = 20
K = NEG + 1
NC = 2
NS = 16
NW = NC * NS
BPW = B // NW


def _tr_body(x_ref, o_ref):
    o_ref[...] = x_ref[...].T


def _transpose(w_out):
    v = w_out.shape[1]
    cb = 1024
    grid = (pl.cdiv(v, cb),)
    return pl.pallas_call(
        _tr_body,
        grid=grid,
        in_specs=[pl.BlockSpec((D, cb), lambda i: (0, i))],
        out_specs=pl.BlockSpec((cb, D), lambda i: (i, 0)),
        out_shape=jax.ShapeDtypeStruct((v, D), jnp.float32),
    )(w_out)


_MESH = plsc.VectorSubcoreMesh(core_axis_name="c", subcore_axis_name="s")


@functools.partial(
    pl.kernel,
    mesh=_MESH,
    compiler_params=pltpu.CompilerParams(use_tc_tiling_on_sc=False),
    out_type=jax.ShapeDtypeStruct((K, B), jnp.float32),
    scratch_types=[
        pltpu.VMEM((BPW,), jnp.int32),
        pltpu.VMEM((BPW,), jnp.int32),
        pltpu.VMEM((BPW, D), jnp.float32),
        pltpu.VMEM((BPW, D), jnp.float32),
        pltpu.VMEM((BPW,), jnp.float32),
        pltpu.SemaphoreType.DMA,
    ],
)
def _sc_score(idx_hbm, cen_hbm, win_hbm, wt_hbm, out_hbm,
              cidx_v, idx_v, h_v, w_v, s_v, sem):
    wid = lax.axis_index("s") * NC + lax.axis_index("c")
    base = wid * BPW

    pltpu.sync_copy(cen_hbm.at[pl.ds(base, BPW)], cidx_v)
    pltpu.async_copy(win_hbm.at[cidx_v], h_v, sem).wait()

    lane = lax.iota(jnp.int32, 16)
    perm_idx = [lane ^ sh for sh in (1, 2, 4, 8)]

    dn = lax.GatherDimensionNumbers(
        offset_dims=(), collapsed_slice_dims=(0,), start_index_map=(0,))

    def hsum(x):
        for idx in perm_idx:
            x = x + lax.gather(x, idx[:, None], dn, (1,),
                               mode=lax.GatherScatterMode.PROMISE_IN_BOUNDS)
        return x

    def per_k(k, carry):
        pltpu.sync_copy(idx_hbm.at[k, pl.ds(base, BPW)], idx_v)
        pltpu.async_copy(wt_hbm.at[idx_v], w_v, sem).wait()

        def per_g(g, c):
            svec = jnp.zeros((16,), jnp.float32)
            for l in range(16):
                i = g * 16 + l
                acc = h_v[i, pl.ds(0, 16)] * w_v[i, pl.ds(0, 16)]
                for j in range(1, D // 16):
                    acc = acc + h_v[i, pl.ds(16 * j, 16)] * w_v[i, pl.ds(16 * j, 16)]
                svec = jnp.where(lane == l, hsum(acc), svec)
            s_v[pl.ds(g * 16, 16)] = svec
            return c

        lax.fori_loop(0, BPW // 16, per_g, 0)
        pltpu.sync_copy(s_v, out_hbm.at[k, pl.ds(base, BPW)])
        return carry

    lax.fori_loop(0, K, per_k, 0)


def kernel(centers, pos, negs, W_in, W_out):
    wt = _transpose(W_out)
    idx_all = jnp.concatenate(
        [pos[None, :].astype(jnp.int32), negs.T.astype(jnp.int32)], axis=0)
    s_all = _sc_score(idx_all, centers.astype(jnp.int32), W_in, wt)
    return s_all[0], s_all[1:].T
